# baseline (speedup 1.0000x reference)
import functools

import jax
import jax.numpy as jnp
from jax import lax
from jax.experimental import pallas as pl
from jax.experimental.pallas import tpu as pltpu
from jax.experimental.pallas import tpu_sc as plsc

BATCH = 4096
HIST = 200
EMBED = 64
N = BATCH * HIST
NC, NS = 2, 16
NW = NC * NS
PER_W = N // NW
CH = 128
NCHUNK = PER_W // CH


def _sc_gather(idx_flat, table):
    mesh = plsc.VectorSubcoreMesh(core_axis_name="c", subcore_axis_name="s")

    @functools.partial(
        pl.kernel,
        out_type=jax.ShapeDtypeStruct((N, EMBED), jnp.float32),
        mesh=mesh,
        scratch_types=[
            pltpu.VMEM((CH,), jnp.int32),
            pltpu.VMEM((CH, EMBED), jnp.float32),
            pltpu.SemaphoreType.DMA,
        ],
        compiler_params=pltpu.CompilerParams(use_tc_tiling_on_sc=False),
    )
    def k(idx_hbm, table_hbm, out_hbm, idx_v, rows_v, sem):
        wid = lax.axis_index("s") * NC + lax.axis_index("c")
        base = wid * PER_W

        def body(j, carry):
            off = base + j * CH
            pltpu.sync_copy(idx_hbm.at[pl.ds(off, CH)], idx_v)
            pltpu.async_copy(table_hbm.at[idx_v], rows_v, sem).wait()
            pltpu.sync_copy(rows_v, out_hbm.at[pl.ds(off, CH)])
            return carry

        lax.fori_loop(0, NCHUNK, body, 0)

    return k(idx_flat, table)


def _mask_body(t_ref, m_ref):
    m_ref[...] = t_ref[...] == 0


def _tc_mask(term):
    blk = 256
    return pl.pallas_call(
        _mask_body,
        out_shape=jax.ShapeDtypeStruct((BATCH, HIST), jnp.bool_),
        in_specs=[pl.BlockSpec((blk, HIST), lambda i: (i, 0))],
        out_specs=pl.BlockSpec((blk, HIST), lambda i: (i, 0)),
        grid=(BATCH // blk,),
    )(term)


def kernel(term, table):
    idx_flat = term.reshape(N)
    rows = _sc_gather(idx_flat, table)
    embedded = rows.reshape(BATCH, HIST, EMBED)
    mask = _tc_mask(term)
    return (embedded, mask)


# speedup vs baseline: 1.1941x; 1.1941x over previous
import functools

import jax
import jax.numpy as jnp
from jax import lax
from jax.experimental import pallas as pl
from jax.experimental.pallas import tpu as pltpu
from jax.experimental.pallas import tpu_sc as plsc

BATCH = 4096
HIST = 200
EMBED = 64
N = BATCH * HIST
NC, NS = 2, 16
NW = NC * NS
PER_W = N // NW
CH = 512
NCHUNK = PER_W // CH
PAIRS = NCHUNK // 2


def _sc_gather(idx_flat, table):
    mesh = plsc.VectorSubcoreMesh(core_axis_name="c", subcore_axis_name="s")

    @functools.partial(
        pl.kernel,
        out_type=jax.ShapeDtypeStruct((N, EMBED), jnp.float32),
        mesh=mesh,
        scratch_types=[
            pltpu.VMEM((CH,), jnp.int32),
            pltpu.VMEM((CH,), jnp.int32),
            pltpu.VMEM((CH, EMBED), jnp.float32),
            pltpu.VMEM((CH, EMBED), jnp.float32),
            pltpu.SemaphoreType.DMA,
            pltpu.SemaphoreType.DMA,
            pltpu.SemaphoreType.DMA,
            pltpu.SemaphoreType.DMA,
            pltpu.SemaphoreType.DMA,
            pltpu.SemaphoreType.DMA,
        ],
        compiler_params=pltpu.CompilerParams(use_tc_tiling_on_sc=False),
    )
    def k(idx_hbm, table_hbm, out_hbm, idx0, idx1, rows0, rows1,
          isem0, isem1, gsem0, gsem1, wsem0, wsem1):
        wid = lax.axis_index("s") * NC + lax.axis_index("c")
        base = wid * PER_W
        idx_v = (idx0, idx1)
        rows_v = (rows0, rows1)
        isem = (isem0, isem1)
        gsem = (gsem0, gsem1)
        wsem = (wsem0, wsem1)

        def idx_slice(j):
            return idx_hbm.at[pl.ds(base + j * CH, CH)]

        def out_slice(j):
            return out_hbm.at[pl.ds(base + j * CH, CH)]

        def start_idx(j, b):
            pltpu.async_copy(idx_slice(j), idx_v[b], isem[b])

        def wait_idx(b):
            pltpu.make_async_copy(idx_slice(0), idx_v[b], isem[b]).wait()

        def start_gather(b):
            pltpu.async_copy(table_hbm.at[idx_v[b]], rows_v[b], gsem[b])

        def wait_gather(b):
            pltpu.make_async_copy(
                table_hbm.at[pl.ds(0, CH)], rows_v[b], gsem[b]).wait()

        def start_wb(j, b):
            pltpu.async_copy(rows_v[b], out_slice(j), wsem[b])

        def wait_wb(b):
            pltpu.make_async_copy(rows_v[b], out_slice(0), wsem[b]).wait()

        start_idx(0, 0)
        start_idx(1, 1)
        wait_idx(0)
        start_gather(0)

        wait_gather(0)
        start_wb(0, 0)
        start_idx(2, 0)
        wait_idx(1)
        start_gather(1)

        wait_gather(1)
        start_wb(1, 1)
        start_idx(3, 1)
        wait_wb(0)
        wait_idx(0)
        start_gather(0)

        def pair_body(p, carry):
            j0 = 2 * p
            wait_gather(0)
            start_wb(j0, 0)
            start_idx(j0 + 2, 0)
            wait_wb(1)
            wait_idx(1)
            start_gather(1)
            wait_gather(1)
            start_wb(j0 + 1, 1)
            start_idx(j0 + 3, 1)
            wait_wb(0)
            wait_idx(0)
            start_gather(0)
            return carry

        lax.fori_loop(1, PAIRS - 1, pair_body, 0)

        wait_gather(0)
        start_wb(NCHUNK - 2, 0)
        wait_wb(1)
        wait_idx(1)
        start_gather(1)

        wait_gather(1)
        start_wb(NCHUNK - 1, 1)

        wait_wb(0)
        wait_wb(1)

    return k(idx_flat, table)


def _mask_body(t_ref, m_ref):
    m_ref[...] = t_ref[...] == 0


def _tc_mask(term):
    blk = 256
    return pl.pallas_call(
        _mask_body,
        out_shape=jax.ShapeDtypeStruct((BATCH, HIST), jnp.bool_),
        in_specs=[pl.BlockSpec((blk, HIST), lambda i: (i, 0))],
        out_specs=pl.BlockSpec((blk, HIST), lambda i: (i, 0)),
        grid=(BATCH // blk,),
    )(term)


def kernel(term, table):
    idx_flat = term.reshape(N)
    rows = _sc_gather(idx_flat, table)
    embedded = rows.reshape(BATCH, HIST, EMBED)
    mask = _tc_mask(term)
    return (embedded, mask)


# speedup vs baseline: 1.7750x; 1.4864x over previous
import functools

import jax
import jax.numpy as jnp
from jax import lax
from jax.experimental import pallas as pl
from jax.experimental.pallas import tpu as pltpu
from jax.experimental.pallas import tpu_sc as plsc

BATCH = 4096
HIST = 200
EMBED = 64
N = BATCH * HIST
NC, NS = 2, 16
NW = NC * NS
PER_W = N // NW
CH = 256
NCHUNK = PER_W // CH
PAIRS = NCHUNK // 2


def _sc_gather(idx_flat, table):
    mesh = plsc.VectorSubcoreMesh(core_axis_name="c", subcore_axis_name="s")

    @functools.partial(
        pl.kernel,
        out_type=jax.ShapeDtypeStruct((N, EMBED), jnp.float32),
        mesh=mesh,
        scratch_types=[
            pltpu.VMEM((CH,), jnp.int32),
            pltpu.VMEM((CH,), jnp.int32),
            pltpu.VMEM((CH, EMBED), jnp.float32),
            pltpu.VMEM((CH, EMBED), jnp.float32),
            pltpu.SemaphoreType.DMA,
            pltpu.SemaphoreType.DMA,
            pltpu.SemaphoreType.DMA,
            pltpu.SemaphoreType.DMA,
            pltpu.SemaphoreType.DMA,
            pltpu.SemaphoreType.DMA,
        ],
    )
    def k(idx_hbm, table_hbm, out_hbm, idx0, idx1, rows0, rows1,
          isem0, isem1, gsem0, gsem1, wsem0, wsem1):
        wid = lax.axis_index("s") * NC + lax.axis_index("c")
        base = wid * PER_W
        idx_s = (idx0, idx1)
        rows_v = (rows0, rows1)
        isem = (isem0, isem1)
        gsem = (gsem0, gsem1)
        wsem = (wsem0, wsem1)

        def start_idx(c, b):
            pltpu.async_copy(
                idx_hbm.at[pl.ds(base + c * CH, CH)], idx_s[b], isem[b])

        def wait_idx(b):
            pltpu.make_async_copy(
                idx_hbm.at[pl.ds(0, CH)], idx_s[b], isem[b]).wait()

        def enqueue_rows(b):
            def group(g, carry):
                vec = idx_s[b][pl.ds(g * 16, 16)]
                for j in range(16):
                    i = vec[j]
                    pltpu.async_copy(table_hbm.at[i],
                                     rows_v[b].at[g * 16 + j], gsem[b])
                return carry

            lax.fori_loop(0, CH // 16, group, 0)

        def drain_rows(b):
            def wgroup(g, carry):
                for j in range(16):
                    pltpu.make_async_copy(
                        table_hbm.at[0], rows_v[b].at[g * 16 + j],
                        gsem[b]).wait()
                return carry

            lax.fori_loop(0, CH // 16, wgroup, 0)

        def start_wb(c, b):
            pltpu.async_copy(
                rows_v[b], out_hbm.at[pl.ds(base + c * CH, CH)], wsem[b])

        def wait_wb(b):
            pltpu.make_async_copy(
                rows_v[b], out_hbm.at[pl.ds(0, CH)], wsem[b]).wait()

        start_idx(0, 0)
        start_idx(1, 1)
        wait_idx(0)
        enqueue_rows(0)

        drain_rows(0)
        start_wb(0, 0)
        start_idx(2, 0)
        wait_idx(1)
        enqueue_rows(1)

        drain_rows(1)
        start_wb(1, 1)
        start_idx(3, 1)
        wait_wb(0)
        wait_idx(0)
        enqueue_rows(0)

        def pair_body(p, carry):
            c0 = 2 * p
            drain_rows(0)
            start_wb(c0, 0)
            start_idx(c0 + 2, 0)
            wait_wb(1)
            wait_idx(1)
            enqueue_rows(1)

            drain_rows(1)
            start_wb(c0 + 1, 1)
            start_idx(c0 + 3, 1)
            wait_wb(0)
            wait_idx(0)
            enqueue_rows(0)
            return carry

        lax.fori_loop(1, PAIRS - 1, pair_body, 0)

        drain_rows(0)
        start_wb(NCHUNK - 2, 0)
        wait_wb(1)
        wait_idx(1)
        enqueue_rows(1)

        drain_rows(1)
        start_wb(NCHUNK - 1, 1)

        wait_wb(0)
        wait_wb(1)

    return k(idx_flat, table)


def _mask_body(t_ref, m_ref):
    m_ref[...] = t_ref[...] == 0


def _tc_mask(term):
    blk = 256
    return pl.pallas_call(
        _mask_body,
        out_shape=jax.ShapeDtypeStruct((BATCH, HIST), jnp.bool_),
        in_specs=[pl.BlockSpec((blk, HIST), lambda i: (i, 0))],
        out_specs=pl.BlockSpec((blk, HIST), lambda i: (i, 0)),
        grid=(BATCH // blk,),
    )(term)


def kernel(term, table):
    idx_flat = term.reshape(N)
    rows = _sc_gather(idx_flat, table)
    embedded = rows.reshape(BATCH, HIST, EMBED)
    mask = _tc_mask(term)
    return (embedded, mask)
